# baseline (speedup 1.0000x reference)
import functools

import jax
import jax.numpy as jnp
from jax import lax
from jax.experimental import pallas as pl
from jax.experimental.pallas import tpu as pltpu
from jax.experimental.pallas import tpu_sc as plsc


def _sc_kernel_fn(n, m, f):
    nw = 32
    njmax = (m + nw - 1) // nw
    ni_chunks = n // 128
    fg = f // 8
    mesh = plsc.VectorSubcoreMesh(core_axis_name="c", subcore_axis_name="s")

    @functools.partial(
        pl.kernel,
        mesh=mesh,
        out_type=jax.ShapeDtypeStruct((m, fg, ni_chunks, 8, 128), jnp.float32),
        compiler_params=pltpu.CompilerParams(needs_layout_passes=False),
        scratch_types=[
            pltpu.VMEM((1024,), jnp.float32),
            pltpu.VMEM((ni_chunks, 128), jnp.int32),
            pltpu.VMEM((ni_chunks, 8, 128), jnp.float32),
        ],
    )
    def sc_kernel(xs_hbm, w_hbm, out_hbm, w_v, idx_v, buf_v):
        wid = lax.axis_index("s") * 2 + lax.axis_index("c")
        pltpu.sync_copy(w_hbm, w_v)

        def j_step(t, carry):
            j = wid + nw * t

            @pl.when(j < m)
            def _():
                pltpu.sync_copy(xs_hbm.at[j], idx_v)
                for g in range(fg):
                    def ic_step(ic, c2):
                        for fr in range(8):
                            col = jnp.full((16,), g * 8 + fr, jnp.int32)
                            for k in range(8):
                                rows = idx_v[ic, pl.ds(k * 16, 16)]
                                vals = plsc.load_gather(
                                    w_v, [rows * 128 + col])
                                buf_v[ic, fr, pl.ds(k * 16, 16)] = vals
                        return c2
                    lax.fori_loop(0, ni_chunks, ic_step, 0)
                    pltpu.sync_copy(buf_v, out_hbm.at[j, g])
            return carry

        lax.fori_loop(0, njmax, j_step, 0)

    return sc_kernel


def kernel(X, W):
    n, m = X.shape
    f = W.shape[1]
    xs = X.astype(jnp.int32).T.reshape(m, n // 128, 128)
    w128 = jnp.zeros((8, 128), jnp.float32).at[:5, :f].set(W).reshape(1024)
    a = _sc_kernel_fn(n, m, f)(xs, w128)
    return a.transpose(2, 4, 0, 1, 3).reshape(n, m, f)


# speedup vs baseline: 32.6900x; 32.6900x over previous
import jax
import jax.numpy as jnp
from jax.experimental import pallas as pl

_J_PER_BLOCK = 8


def _tc_body(xt_ref, wt8_ref, o_ref):
    jb = xt_ref.shape[0]
    ni = xt_ref.shape[1]
    xt = xt_ref[...].astype(jnp.int32)
    wt8 = wt8_ref[...]
    iota8 = jax.lax.broadcasted_iota(jnp.int32, (8, ni), 0)
    for j in range(jb):
        oh = jnp.where(iota8 == xt[j][None, :], 1.0, 0.0)
        o_ref[j] = jax.lax.dot_general(
            wt8, oh, (((1,), (0,)), ((), ())),
            preferred_element_type=jnp.float32)


def kernel(X, W):
    n, m = X.shape
    f = W.shape[1]
    jb = _J_PER_BLOCK
    xt = X.astype(jnp.int32).T
    wt8 = jnp.zeros((f, 8), jnp.float32).at[:, :5].set(W.T)
    outt = pl.pallas_call(
        _tc_body,
        grid=(m // jb,),
        in_specs=[
            pl.BlockSpec((jb, n), lambda i: (i, 0)),
            pl.BlockSpec((f, 8), lambda i: (0, 0)),
        ],
        out_specs=pl.BlockSpec((jb, f, n), lambda i: (i, 0, 0)),
        out_shape=jax.ShapeDtypeStruct((m, f, n), jnp.float32),
    )(xt, wt8)
    return outt.transpose(2, 0, 1)
